# baseline (speedup 1.0000x reference)
import functools

import jax
import jax.numpy as jnp
from jax import lax
from jax.experimental import pallas as pl
from jax.experimental.pallas import tpu as pltpu
from jax.experimental.pallas import tpu_sc as plsc

EMB = 64
MAX_LEN = 200
BATCH = 4096
VOCAB = 1000000
ROWPAIR = 128
ZROWS = VOCAB * EMB // ROWPAIR

NC = 2
NS = 16
NW = NC * NS
BW = BATCH // NW
NBUF = 4
NOBUF = 2
LANES = 16
ER = EMB // 8
GRP = 4

VCHUNK = 128
NFULL = VOCAB // VCHUNK
ACH_BASE = NFULL // NW
ACH_EXTRA = NFULL - ACH_BASE * NW
ABUF = 2

_mesh = plsc.VectorSubcoreMesh(core_axis_name="c", subcore_axis_name="s")


@functools.partial(
    pl.kernel,
    mesh=_mesh,
    out_type=jax.ShapeDtypeStruct((ZROWS, ROWPAIR), jnp.float32),
    compiler_params=pltpu.CompilerParams(use_tc_tiling_on_sc=True,
                                         needs_layout_passes=False),
    scratch_types=[
        [pltpu.VMEM((EMB, VCHUNK), jnp.float32)] * ABUF,
        [pltpu.VMEM((VCHUNK // 2, ROWPAIR), jnp.float32)] * ABUF,
        [pltpu.SemaphoreType.DMA] * ABUF,
        [pltpu.SemaphoreType.DMA] * ABUF,
    ],
)
def _repack(tabt_hbm, tail_hbm, z_hbm, ibufs, obufs, isems, osems):
    wid = lax.axis_index("s") * NC + lax.axis_index("c")
    iota = lax.broadcasted_iota(jnp.int32, (LANES,), 0)
    zero16 = jnp.broadcast_to(0, (LANES,))
    rows_out = [lax.shift_left(iota + (c0 * LANES), 6) for c0 in range(8)]

    my_n = ACH_BASE + jnp.where(wid < ACH_EXTRA, 1, 0)

    def stage(k, slot):
        c = k * NW + wid
        src = tabt_hbm.at[:, pl.ds(c * VCHUNK, VCHUNK)]
        return pltpu.make_async_copy(src, ibufs[slot], isems[slot])

    def zout(k, slot):
        c = k * NW + wid
        dst = z_hbm.at[pl.ds(c * (VCHUNK // 2), VCHUNK // 2)]
        return pltpu.make_async_copy(obufs[slot], dst, osems[slot])

    def transpose_chunk(ibuf, obuf, width):
        def body(k, carry):
            diag = (iota + k) & 15
            for e0 in range(0, EMB, LANES):
                cols = diag + e0
                cshift = lax.shift_left(cols, 7)
                for c0 in range(width // LANES):
                    v = plsc.load_gather(ibuf, [zero16, cshift + iota + c0 * LANES])
                    plsc.store_scatter(obuf, [zero16, rows_out[c0] + cols], v)
            return carry

        lax.fori_loop(0, LANES, body, 0)

    def substep(k, slot):
        stage(k, slot).wait()

        @pl.when(k >= ABUF)
        def _():
            zout(k - ABUF, slot).wait()

        transpose_chunk(ibufs[slot], obufs[slot], VCHUNK)
        zout(k, slot).start()

        @pl.when(k + ABUF < my_n)
        def _():
            stage(k + ABUF, slot).start()

    stage(0, 0).start()
    stage(1, 1).start()

    def body(k2, carry):
        substep(k2 * ABUF, 0)
        substep(k2 * ABUF + 1, 1)
        return carry

    lax.fori_loop(0, my_n // ABUF, body, 0)

    @pl.when(my_n % ABUF == 1)
    def _():
        substep(my_n - 1, 0)

    zout(0, 0).wait()
    zout(0, 1).wait()

    @pl.when(wid == 0)
    def _():
        pltpu.sync_copy(tail_hbm, ibufs[0])
        transpose_chunk(ibufs[0], obufs[0], EMB)
        pltpu.sync_copy(obufs[0].at[pl.ds(0, 32)],
                        z_hbm.at[pl.ds(ZROWS - 32, 32)])


@functools.partial(
    pl.kernel,
    mesh=_mesh,
    out_type=jax.ShapeDtypeStruct((MAX_LEN, ER, NW, 8, 128), jnp.float32),
    compiler_params=pltpu.CompilerParams(use_tc_tiling_on_sc=False,
                                         needs_layout_passes=False),
    scratch_types=[
        pltpu.VMEM((MAX_LEN, BW), jnp.int32),
        pltpu.VMEM((MAX_LEN, EMB), jnp.float32),
        [pltpu.VMEM((BW, ROWPAIR), jnp.float32)] * NBUF,
        [pltpu.VMEM((ER, 8, BW), jnp.float32)] * NOBUF,
        [pltpu.VMEM((BW,), jnp.int32)] * NBUF,
        [pltpu.SemaphoreType.DMA] * NBUF,
        [pltpu.SemaphoreType.DMA] * NOBUF,
    ],
)
def _emb_lookup(tok_hbm, table_hbm, pos_hbm, out_hbm,
                idx_v, pos_v, gbufs, obufs, ibufs, gsems, osems):
    wid = lax.axis_index("s") * NC + lax.axis_index("c")

    pltpu.sync_copy(pos_hbm, pos_v)
    pltpu.sync_copy(tok_hbm.at[:, pl.ds(wid * BW, BW)], idx_v)

    iota = lax.broadcasted_iota(jnp.int32, (LANES,), 0)
    zero16 = jnp.broadcast_to(0, (LANES,))
    rows_c = [iota + (b0 * LANES) for b0 in range(BW // LANES)]
    rows_g = [lax.shift_left(iota + (b0 * LANES), 7) for b0 in range(BW // LANES)]

    def prep_gather(l, slot):
        for b0 in range(BW // LANES):
            tv = idx_v[l, pl.ds(b0 * LANES, LANES)]
            ibufs[slot][pl.ds(b0 * LANES, LANES)] = lax.shift_right_logical(tv, 1)
        pltpu.make_async_copy(table_hbm.at[ibufs[slot]], gbufs[slot],
                              gsems[slot]).start()

    def gather_wait(slot):
        pltpu.make_async_copy(table_hbm.at[ibufs[slot]], gbufs[slot],
                              gsems[slot]).wait()

    def out_dma(l, oslot):
        return pltpu.make_async_copy(obufs[oslot], out_hbm.at[l, :, wid],
                                     osems[oslot])

    def substep(l, slot, oslot):
        gather_wait(slot)
        gbuf = gbufs[slot]
        obuf = obufs[oslot]

        @pl.when(l >= NOBUF)
        def _():
            out_dma(l - NOBUF, oslot).wait()

        lbase = jnp.broadcast_to(l * EMB, (LANES,))
        gbase = [rows_g[b0] + lax.shift_left(
                     idx_v[l, pl.ds(b0 * LANES, LANES)] & 1, 6)
                 for b0 in range(BW // LANES)]

        def trans_k(k, carry):
            diag = (iota + k) & 15
            for e0 in range(0, EMB, LANES):
                cols = diag + e0
                p = plsc.load_gather(pos_v, [zero16, lbase + cols])
                cshift = lax.shift_left(cols, 7)
                for b0 in range(BW // LANES):
                    v = plsc.load_gather(gbuf,
                                         [zero16, gbase[b0] + cols])
                    plsc.store_scatter(obuf, [zero16, zero16,
                                              cshift + rows_c[b0]], v + p)
            return carry

        lax.fori_loop(0, LANES, trans_k, 0)

        @pl.when(l + NBUF < MAX_LEN)
        def _():
            prep_gather(l + NBUF, slot)

        out_dma(l, oslot).start()

    for s in range(NBUF):
        prep_gather(s, s)

    def body(k, carry):
        for s in range(GRP):
            substep(k * GRP + s, s % NBUF, s % NOBUF)
        return carry

    lax.fori_loop(0, MAX_LEN // GRP, body, 0)

    for l in range(MAX_LEN - NOBUF, MAX_LEN):
        out_dma(l, l % NOBUF).wait()


def kernel(tokens, token_table, pos_emb):
    tok_t = tokens.T
    tab_t = token_table.T
    tail = jnp.pad(token_table[NFULL * VCHUNK:].T,
                   ((0, 0), (0, ROWPAIR - EMB)))
    z = _repack(tab_t, tail)
    o5 = _emb_lookup(tok_t, z, pos_emb)
    return o5.transpose(2, 4, 0, 1, 3).reshape(BATCH, MAX_LEN, EMB)


# speedup vs baseline: 2.1226x; 2.1226x over previous
import functools

import jax
import jax.numpy as jnp
from jax import lax
from jax.experimental import pallas as pl
from jax.experimental.pallas import tpu as pltpu
from jax.experimental.pallas import tpu_sc as plsc

EMB = 64
MAX_LEN = 200
BATCH = 4096
VOCAB = 1000000
ROWPAIR = 128
ZROWS = VOCAB * EMB // ROWPAIR

NC = 2
NS = 16
NW = NC * NS
BW = BATCH // NW
NBUF = 4
NOBUF = 2
LANES = 16
ER = EMB // 8
GRP = 4

VCHUNK = 128
NFULL = VOCAB // VCHUNK
ACH_BASE = NFULL // NW
ACH_EXTRA = NFULL - ACH_BASE * NW
ABUF = 2

_mesh = plsc.VectorSubcoreMesh(core_axis_name="c", subcore_axis_name="s")


@functools.partial(
    pl.kernel,
    mesh=_mesh,
    out_type=jax.ShapeDtypeStruct((ZROWS, ROWPAIR), jnp.float32),
    compiler_params=pltpu.CompilerParams(use_tc_tiling_on_sc=True,
                                         needs_layout_passes=False),
    scratch_types=[
        [pltpu.VMEM((EMB, VCHUNK), jnp.float32)] * ABUF,
        [pltpu.VMEM((VCHUNK // 2, ROWPAIR), jnp.float32)] * ABUF,
        [pltpu.SemaphoreType.DMA] * ABUF,
        [pltpu.SemaphoreType.DMA] * ABUF,
    ],
)
def _repack(tabt_hbm, tail_hbm, z_hbm, ibufs, obufs, isems, osems):
    wid = lax.axis_index("s") * NC + lax.axis_index("c")
    iota = lax.broadcasted_iota(jnp.int32, (LANES,), 0)
    zero16 = jnp.broadcast_to(0, (LANES,))
    rows_out = [lax.shift_left(iota + (c0 * LANES), 6) for c0 in range(8)]

    my_n = ACH_BASE + jnp.where(wid < ACH_EXTRA, 1, 0)

    def stage(k, slot):
        c = k * NW + wid
        src = tabt_hbm.at[:, pl.ds(c * VCHUNK, VCHUNK)]
        return pltpu.make_async_copy(src, ibufs[slot], isems[slot])

    def zout(k, slot):
        c = k * NW + wid
        dst = z_hbm.at[pl.ds(c * (VCHUNK // 2), VCHUNK // 2)]
        return pltpu.make_async_copy(obufs[slot], dst, osems[slot])

    def transpose_chunk(ibuf, obuf, width):
        def body(k, carry):
            diag = (iota + k) & 15
            for e0 in range(0, EMB, LANES):
                cols = diag + e0
                cshift = lax.shift_left(cols, 7)
                vs = [plsc.load_gather(ibuf, [zero16, cshift + iota + c0 * LANES])
                      for c0 in range(width // LANES)]
                for c0 in range(width // LANES):
                    plsc.store_scatter(obuf, [zero16, rows_out[c0] + cols], vs[c0])
            return carry

        lax.fori_loop(0, LANES, body, 0)

    def substep(k, slot):
        stage(k, slot).wait()

        @pl.when(k >= ABUF)
        def _():
            zout(k - ABUF, slot).wait()

        transpose_chunk(ibufs[slot], obufs[slot], VCHUNK)
        zout(k, slot).start()

        @pl.when(k + ABUF < my_n)
        def _():
            stage(k + ABUF, slot).start()

    stage(0, 0).start()
    stage(1, 1).start()

    def body(k2, carry):
        substep(k2 * ABUF, 0)
        substep(k2 * ABUF + 1, 1)
        return carry

    lax.fori_loop(0, my_n // ABUF, body, 0)

    @pl.when(my_n % ABUF == 1)
    def _():
        substep(my_n - 1, 0)

    zout(0, 0).wait()
    zout(0, 1).wait()

    @pl.when(wid == 0)
    def _():
        pltpu.sync_copy(tail_hbm, ibufs[0])
        transpose_chunk(ibufs[0], obufs[0], EMB)
        pltpu.sync_copy(obufs[0].at[pl.ds(0, 32)],
                        z_hbm.at[pl.ds(ZROWS - 32, 32)])


@functools.partial(
    pl.kernel,
    mesh=_mesh,
    out_type=jax.ShapeDtypeStruct((MAX_LEN, ER, NW, 8, 128), jnp.float32),
    compiler_params=pltpu.CompilerParams(use_tc_tiling_on_sc=False,
                                         needs_layout_passes=False),
    scratch_types=[
        pltpu.VMEM((MAX_LEN, BW), jnp.int32),
        pltpu.VMEM((MAX_LEN, EMB), jnp.float32),
        [pltpu.VMEM((BW, ROWPAIR), jnp.float32)] * NBUF,
        [pltpu.VMEM((ER, 8, BW), jnp.float32)] * NOBUF,
        [pltpu.VMEM((BW,), jnp.int32)] * NBUF,
        [pltpu.SemaphoreType.DMA] * NBUF,
        [pltpu.SemaphoreType.DMA] * NOBUF,
    ],
)
def _emb_lookup(tok_hbm, table_hbm, pos_hbm, out_hbm,
                idx_v, pos_v, gbufs, obufs, ibufs, gsems, osems):
    wid = lax.axis_index("s") * NC + lax.axis_index("c")

    pltpu.sync_copy(pos_hbm, pos_v)
    pltpu.sync_copy(tok_hbm.at[:, pl.ds(wid * BW, BW)], idx_v)

    iota = lax.broadcasted_iota(jnp.int32, (LANES,), 0)
    zero16 = jnp.broadcast_to(0, (LANES,))
    rows_c = [iota + (b0 * LANES) for b0 in range(BW // LANES)]
    rows_g = [lax.shift_left(iota + (b0 * LANES), 7) for b0 in range(BW // LANES)]

    def prep_gather(l, slot):
        for b0 in range(BW // LANES):
            tv = idx_v[l, pl.ds(b0 * LANES, LANES)]
            ibufs[slot][pl.ds(b0 * LANES, LANES)] = lax.shift_right_logical(tv, 1)
        pltpu.make_async_copy(table_hbm.at[ibufs[slot]], gbufs[slot],
                              gsems[slot]).start()

    def gather_wait(slot):
        pltpu.make_async_copy(table_hbm.at[ibufs[slot]], gbufs[slot],
                              gsems[slot]).wait()

    def out_dma(l, oslot):
        return pltpu.make_async_copy(obufs[oslot], out_hbm.at[l, :, wid],
                                     osems[oslot])

    def substep(l, slot, oslot):
        gather_wait(slot)
        gbuf = gbufs[slot]
        obuf = obufs[oslot]

        @pl.when(l >= NOBUF)
        def _():
            out_dma(l - NOBUF, oslot).wait()

        lbase = jnp.broadcast_to(l * EMB, (LANES,))
        gbase = [rows_g[b0] + lax.shift_left(
                     idx_v[l, pl.ds(b0 * LANES, LANES)] & 1, 6)
                 for b0 in range(BW // LANES)]

        def trans_k(k, carry):
            diag = (iota + k) & 15
            for e0 in range(0, EMB, LANES):
                cols = diag + e0
                p = plsc.load_gather(pos_v, [zero16, lbase + cols])
                cshift = lax.shift_left(cols, 7)
                vs = [plsc.load_gather(gbuf, [zero16, gbase[b0] + cols])
                      for b0 in range(BW // LANES)]
                for b0 in range(BW // LANES):
                    plsc.store_scatter(obuf, [zero16, zero16,
                                              cshift + rows_c[b0]], vs[b0] + p)
            return carry

        lax.fori_loop(0, LANES, trans_k, 0)

        @pl.when(l + NBUF < MAX_LEN)
        def _():
            prep_gather(l + NBUF, slot)

        out_dma(l, oslot).start()

    for s in range(NBUF):
        prep_gather(s, s)

    def body(k, carry):
        for s in range(GRP):
            substep(k * GRP + s, s % NBUF, s % NOBUF)
        return carry

    lax.fori_loop(0, MAX_LEN // GRP, body, 0)

    for l in range(MAX_LEN - NOBUF, MAX_LEN):
        out_dma(l, l % NOBUF).wait()


def kernel(tokens, token_table, pos_emb):
    tok_t = tokens.T
    tab_t = token_table.T
    tail = jnp.pad(token_table[NFULL * VCHUNK:].T,
                   ((0, 0), (0, ROWPAIR - EMB)))
    z = _repack(tab_t, tail)
    o5 = _emb_lookup(tok_t, z, pos_emb)
    return o5.transpose(2, 4, 0, 1, 3).reshape(BATCH, MAX_LEN, EMB)


# speedup vs baseline: 2.3509x; 1.1075x over previous
import functools

import jax
import jax.numpy as jnp
from jax import lax
from jax.experimental import pallas as pl
from jax.experimental.pallas import tpu as pltpu
from jax.experimental.pallas import tpu_sc as plsc

EMB = 64
MAX_LEN = 200
BATCH = 4096
VOCAB = 1000000
ROWPAIR = 128
ZROWS = VOCAB * EMB // ROWPAIR

NC = 2
NS = 16
NW = NC * NS
BW = BATCH // NW
NBUF = 4
NOBUF = 2
LANES = 16
ER = EMB // 8
GRP = 4

VCHUNK = 128
NFULL = VOCAB // VCHUNK
ACH_BASE = NFULL // NW
ACH_EXTRA = NFULL - ACH_BASE * NW
ABUF = 3

_mesh = plsc.VectorSubcoreMesh(core_axis_name="c", subcore_axis_name="s")


@functools.partial(
    pl.kernel,
    mesh=_mesh,
    out_type=jax.ShapeDtypeStruct((ZROWS, ROWPAIR), jnp.float32),
    compiler_params=pltpu.CompilerParams(use_tc_tiling_on_sc=True,
                                         needs_layout_passes=False),
    scratch_types=[
        [pltpu.VMEM((EMB, VCHUNK), jnp.float32)] * ABUF,
        [pltpu.VMEM((VCHUNK // 2, ROWPAIR), jnp.float32)] * ABUF,
        [pltpu.SemaphoreType.DMA] * ABUF,
        [pltpu.SemaphoreType.DMA] * ABUF,
    ],
)
def _repack(tabt_hbm, tail_hbm, z_hbm, ibufs, obufs, isems, osems):
    wid = lax.axis_index("s") * NC + lax.axis_index("c")
    iota = lax.broadcasted_iota(jnp.int32, (LANES,), 0)
    zero16 = jnp.broadcast_to(0, (LANES,))
    rows_out = [lax.shift_left(iota + (c0 * LANES), 6) for c0 in range(8)]

    my_n = ACH_BASE + jnp.where(wid < ACH_EXTRA, 1, 0)

    def stage(k, slot):
        c = k * NW + wid
        src = tabt_hbm.at[:, pl.ds(c * VCHUNK, VCHUNK)]
        return pltpu.make_async_copy(src, ibufs[slot], isems[slot])

    def zout(k, slot):
        c = k * NW + wid
        dst = z_hbm.at[pl.ds(c * (VCHUNK // 2), VCHUNK // 2)]
        return pltpu.make_async_copy(obufs[slot], dst, osems[slot])

    def transpose_chunk(ibuf, obuf, width):
        def body(k, carry):
            diag = (iota + k) & 15
            for e0 in range(0, EMB, LANES):
                cols = diag + e0
                cshift = lax.shift_left(cols, 7)
                vs = [plsc.load_gather(ibuf, [zero16, cshift + iota + c0 * LANES])
                      for c0 in range(width // LANES)]
                for c0 in range(width // LANES):
                    plsc.store_scatter(obuf, [zero16, rows_out[c0] + cols], vs[c0])
            return carry

        lax.fori_loop(0, LANES, body, 0)

    def substep(k, slot):
        stage(k, slot).wait()

        @pl.when(k >= ABUF)
        def _():
            zout(k - ABUF, slot).wait()

        transpose_chunk(ibufs[slot], obufs[slot], VCHUNK)
        zout(k, slot).start()

        @pl.when(k + ABUF < my_n)
        def _():
            stage(k + ABUF, slot).start()

    for s0 in range(ABUF):
        stage(s0, s0).start()

    def body(k2, carry):
        for s0 in range(ABUF):
            substep(k2 * ABUF + s0, s0)
        return carry

    lax.fori_loop(0, ACH_BASE // ABUF, body, 0)

    substep(ACH_BASE - 1, (ACH_BASE - 1) % ABUF)

    @pl.when(my_n > ACH_BASE)
    def _():
        substep(ACH_BASE, ACH_BASE % ABUF)

    for s0 in range(ABUF):
        zout(0, s0).wait()

    @pl.when(wid == 0)
    def _():
        pltpu.sync_copy(tail_hbm, ibufs[0])
        transpose_chunk(ibufs[0], obufs[0], EMB)
        pltpu.sync_copy(obufs[0].at[pl.ds(0, 32)],
                        z_hbm.at[pl.ds(ZROWS - 32, 32)])


@functools.partial(
    pl.kernel,
    mesh=_mesh,
    out_type=jax.ShapeDtypeStruct((MAX_LEN, ER, NW, 8, 128), jnp.float32),
    compiler_params=pltpu.CompilerParams(use_tc_tiling_on_sc=False,
                                         needs_layout_passes=False),
    scratch_types=[
        pltpu.VMEM((MAX_LEN, BW), jnp.int32),
        pltpu.VMEM((MAX_LEN, EMB), jnp.float32),
        [pltpu.VMEM((BW, ROWPAIR), jnp.float32)] * NBUF,
        [pltpu.VMEM((ER, 8, BW), jnp.float32)] * NOBUF,
        [pltpu.VMEM((BW,), jnp.int32)] * NBUF,
        [pltpu.SemaphoreType.DMA] * NBUF,
        [pltpu.SemaphoreType.DMA] * NOBUF,
    ],
)
def _emb_lookup(tok_hbm, table_hbm, pos_hbm, out_hbm,
                idx_v, pos_v, gbufs, obufs, ibufs, gsems, osems):
    wid = lax.axis_index("s") * NC + lax.axis_index("c")

    pltpu.sync_copy(pos_hbm, pos_v)
    pltpu.sync_copy(tok_hbm.at[:, pl.ds(wid * BW, BW)], idx_v)

    iota = lax.broadcasted_iota(jnp.int32, (LANES,), 0)
    zero16 = jnp.broadcast_to(0, (LANES,))
    rows_c = [iota + (b0 * LANES) for b0 in range(BW // LANES)]
    rows_g = [lax.shift_left(iota + (b0 * LANES), 7) for b0 in range(BW // LANES)]

    def prep_gather(l, slot):
        for b0 in range(BW // LANES):
            tv = idx_v[l, pl.ds(b0 * LANES, LANES)]
            ibufs[slot][pl.ds(b0 * LANES, LANES)] = lax.shift_right_logical(tv, 1)
        pltpu.make_async_copy(table_hbm.at[ibufs[slot]], gbufs[slot],
                              gsems[slot]).start()

    def gather_wait(slot):
        pltpu.make_async_copy(table_hbm.at[ibufs[slot]], gbufs[slot],
                              gsems[slot]).wait()

    def out_dma(l, oslot):
        return pltpu.make_async_copy(obufs[oslot], out_hbm.at[l, :, wid],
                                     osems[oslot])

    def substep(l, slot, oslot):
        gather_wait(slot)
        gbuf = gbufs[slot]
        obuf = obufs[oslot]

        @pl.when(l >= NOBUF)
        def _():
            out_dma(l - NOBUF, oslot).wait()

        lbase = jnp.broadcast_to(l * EMB, (LANES,))
        gbase = [rows_g[b0] + lax.shift_left(
                     idx_v[l, pl.ds(b0 * LANES, LANES)] & 1, 6)
                 for b0 in range(BW // LANES)]

        def trans_k(k, carry):
            diag = (iota + k) & 15
            for e0 in range(0, EMB, LANES):
                cols = diag + e0
                p = plsc.load_gather(pos_v, [zero16, lbase + cols])
                cshift = lax.shift_left(cols, 7)
                vs = [plsc.load_gather(gbuf, [zero16, gbase[b0] + cols])
                      for b0 in range(BW // LANES)]
                for b0 in range(BW // LANES):
                    plsc.store_scatter(obuf, [zero16, zero16,
                                              cshift + rows_c[b0]], vs[b0] + p)
            return carry

        lax.fori_loop(0, LANES, trans_k, 0)

        @pl.when(l + NBUF < MAX_LEN)
        def _():
            prep_gather(l + NBUF, slot)

        out_dma(l, oslot).start()

    for s in range(NBUF):
        prep_gather(s, s)

    def body(k, carry):
        for s in range(GRP):
            substep(k * GRP + s, s % NBUF, s % NOBUF)
        return carry

    lax.fori_loop(0, MAX_LEN // GRP, body, 0)

    for l in range(MAX_LEN - NOBUF, MAX_LEN):
        out_dma(l, l % NOBUF).wait()


def kernel(tokens, token_table, pos_emb):
    tok_t = tokens.T
    tab_t = token_table.T
    tail = jnp.pad(token_table[NFULL * VCHUNK:].T,
                   ((0, 0), (0, ROWPAIR - EMB)))
    z = _repack(tab_t, tail)
    o5 = _emb_lookup(tok_t, z, pos_emb)
    return o5.transpose(2, 4, 0, 1, 3).reshape(BATCH, MAX_LEN, EMB)
